# Initial kernel scaffold; baseline (speedup 1.0000x reference)
import jax
import jax.numpy as jnp
from jax.experimental import pallas as pl


def kernel(input_seqs, table):
    raise NotImplementedError("write your pallas kernel here")


# speedup vs baseline: 7.7360x; 7.7360x over previous
import functools

import jax
import jax.numpy as jnp
from jax import lax
from jax.experimental import pallas as pl
from jax.experimental.pallas import tpu as pltpu
from jax.experimental.pallas import tpu_sc as plsc

_B, _S, _D = 4096, 200, 128
_TOTAL = _B * _S
_NW = 32
_PER_W = _TOTAL // _NW
_CHUNK = 400
_NCHUNK = _PER_W // _CHUNK


def _emb_kernel(idx_hbm, table_hbm, out_hbm, idx_v, rows_v, sem):
    info = plsc.get_sparse_core_info()
    wid = lax.axis_index("s") * info.num_cores + lax.axis_index("c")
    base = wid * _PER_W

    def body(j, carry):
        off = base + j * _CHUNK
        pltpu.sync_copy(idx_hbm.at[pl.ds(off, _CHUNK)], idx_v)
        pltpu.async_copy(table_hbm.at[idx_v], rows_v, sem).wait()
        pltpu.sync_copy(rows_v, out_hbm.at[pl.ds(off, _CHUNK)])
        return carry

    lax.fori_loop(0, _NCHUNK, body, 0)


@jax.jit
def _emb(idx, table):
    mesh = plsc.VectorSubcoreMesh(core_axis_name="c", subcore_axis_name="s")
    run = functools.partial(
        pl.kernel,
        mesh=mesh,
        out_type=jax.ShapeDtypeStruct((_TOTAL, _D), jnp.float32),
        scratch_types=[
            pltpu.VMEM((_CHUNK,), jnp.int32),
            pltpu.VMEM((_CHUNK, _D), jnp.float32),
            pltpu.SemaphoreType.DMA,
        ],
    )(_emb_kernel)
    return run(idx, table)


def kernel(input_seqs, table):
    idx = input_seqs.reshape(_TOTAL).astype(jnp.int32)
    out = _emb(idx, table)
    return out.reshape(_B, _S, _D)


# speedup vs baseline: 9.2571x; 1.1966x over previous
import functools

import jax
import jax.numpy as jnp
from jax import lax
from jax.experimental import pallas as pl
from jax.experimental.pallas import tpu as pltpu
from jax.experimental.pallas import tpu_sc as plsc

_B, _S, _D = 4096, 200, 128
_TOTAL = _B * _S
_NW = 32
_PER_W = _TOTAL // _NW
_CHUNK = 400
_NCHUNK = _PER_W // _CHUNK
_NT = _NCHUNK // 2


def _emb_kernel(idx_hbm, table_hbm, out_hbm, idx0, idx1, rows0, rows1,
                g0, g1, o0, o1):
    info = plsc.get_sparse_core_info()
    wid = lax.axis_index("s") * info.num_cores + lax.axis_index("c")
    base = wid * _PER_W
    idx_v = (idx0, idx1)
    rows_v = (rows0, rows1)
    gsem = (g0, g1)
    osem = (o0, o1)

    def start_gather(b, j):
        off = base + j * _CHUNK
        pltpu.sync_copy(idx_hbm.at[pl.ds(off, _CHUNK)], idx_v[b])
        pltpu.async_copy(table_hbm.at[idx_v[b]], rows_v[b], gsem[b])

    def wait_gather(b):
        pltpu.make_async_copy(table_hbm.at[idx_v[b]], rows_v[b],
                              gsem[b]).wait()

    def start_out(b, j):
        pltpu.async_copy(rows_v[b],
                         out_hbm.at[pl.ds(base + j * _CHUNK, _CHUNK)],
                         osem[b])

    def wait_out(b, j):
        pltpu.make_async_copy(rows_v[b],
                              out_hbm.at[pl.ds(base + j * _CHUNK, _CHUNK)],
                              osem[b]).wait()

    start_gather(0, 0)
    start_gather(1, 1)

    def body(t, carry):
        j0 = 2 * t
        wait_gather(0)
        start_out(0, j0)
        wait_gather(1)
        start_out(1, j0 + 1)

        @pl.when(t < _NT - 1)
        def _():
            wait_out(0, j0)
            start_gather(0, j0 + 2)
            wait_out(1, j0 + 1)
            start_gather(1, j0 + 3)

        return carry

    lax.fori_loop(0, _NT, body, 0)
    wait_out(0, _NCHUNK - 2)
    wait_out(1, _NCHUNK - 1)


@jax.jit
def _emb(idx, table):
    mesh = plsc.VectorSubcoreMesh(core_axis_name="c", subcore_axis_name="s")
    run = functools.partial(
        pl.kernel,
        mesh=mesh,
        out_type=jax.ShapeDtypeStruct((_TOTAL, _D), jnp.float32),
        scratch_types=[
            pltpu.VMEM((_CHUNK,), jnp.int32),
            pltpu.VMEM((_CHUNK,), jnp.int32),
            pltpu.VMEM((_CHUNK, _D), jnp.float32),
            pltpu.VMEM((_CHUNK, _D), jnp.float32),
            pltpu.SemaphoreType.DMA,
            pltpu.SemaphoreType.DMA,
            pltpu.SemaphoreType.DMA,
            pltpu.SemaphoreType.DMA,
        ],
    )(_emb_kernel)
    return run(idx, table)


def kernel(input_seqs, table):
    idx = input_seqs.reshape(_TOTAL).astype(jnp.int32)
    out = _emb(idx, table)
    return out.reshape(_B, _S, _D)
